# baseline (speedup 1.0000x reference)
import functools

import jax
import jax.numpy as jnp
from jax import lax
from jax.experimental import pallas as pl
from jax.experimental.pallas import tpu as pltpu
from jax.experimental.pallas import tpu_sc as plsc

B = 4096
D = 128
NC = 2
NS = 16
L = 16
NW = NC * NS
BPW = B // NW

_mesh = plsc.VectorSubcoreMesh(core_axis_name="c", subcore_axis_name="s")

def _make_sc_gather(batch):
    bpw = batch // NW

    def _sc_gather_body(uid_hbm, iid_hbm, uemb_hbm, iemb_hbm, ubias_hbm,
                        ibias_hbm, inter_out, bias_out,
                        uidx_v, iidx_v, urows, irows, ubv, ibv, sem):
        wid = lax.axis_index("s") * NC + lax.axis_index("c")
        base = wid * bpw
        pltpu.sync_copy(uid_hbm.at[pl.ds(base, bpw)], uidx_v)
        pltpu.sync_copy(iid_hbm.at[pl.ds(base, bpw)], iidx_v)
        cu = pltpu.async_copy(uemb_hbm.at[uidx_v], urows, sem)
        ci = pltpu.async_copy(iemb_hbm.at[iidx_v], irows, sem)
        cub = pltpu.async_copy(ubias_hbm.at[uidx_v], ubv, sem)
        cib = pltpu.async_copy(ibias_hbm.at[iidx_v], ibv, sem)
        cu.wait()
        ci.wait()
        cub.wait()
        cib.wait()

        @plsc.parallel_loop(0, bpw, 1, unroll=2)
        def _mul(r):
            for cc in range(D // L):
                sl = pl.ds(cc * L, L)
                urows[r, sl] = urows[r, sl] * irows[r, sl]

        for k in range(bpw // L):
            sl = pl.ds(k * L, L)
            ubv[sl] = ubv[sl] + ibv[sl]
        pltpu.sync_copy(urows, inter_out.at[pl.ds(base, bpw)])
        pltpu.sync_copy(ubv, bias_out.at[pl.ds(base, bpw)])

    return pl.kernel(
        _sc_gather_body,
        mesh=_mesh,
        out_type=[
            jax.ShapeDtypeStruct((batch, D), jnp.float32),
            jax.ShapeDtypeStruct((batch,), jnp.float32),
        ],
        scratch_types=[
            pltpu.VMEM((bpw,), jnp.int32),
            pltpu.VMEM((bpw,), jnp.int32),
            pltpu.VMEM((bpw, D), jnp.float32),
            pltpu.VMEM((bpw, D), jnp.float32),
            pltpu.VMEM((bpw,), jnp.float32),
            pltpu.VMEM((bpw,), jnp.float32),
            pltpu.SemaphoreType.DMA,
        ],
    )


HB = B // 2
_sc_gather_half = _make_sc_gather(HB)


def _tc_body(inter_ref, bias_ref, gb_ref, w1_ref,
             b1_ref, w2_ref, b2_ref, w3_ref, b3_ref, out_ref):
    x = inter_ref[...]
    cf = jnp.sum(x, axis=1, keepdims=True)
    cf = cf + bias_ref[...].reshape(-1, 1) + gb_ref[0, 0]
    h = jnp.maximum(jnp.dot(x, w1_ref[...], preferred_element_type=jnp.float32)
                    + b1_ref[...], 0.0)
    h = jnp.maximum(jnp.dot(h, w2_ref[...], preferred_element_type=jnp.float32)
                    + b2_ref[...], 0.0)
    mlp = jnp.dot(h, w3_ref[...], preferred_element_type=jnp.float32) + b3_ref[...]
    out_ref[...] = (cf + mlp).reshape(1, -1)


def _tc_call(inter, biasg, global_bias, W1, b1, W2, b2, W3, b3):
    n = inter.shape[0]
    out = pl.pallas_call(
        _tc_body,
        out_shape=jax.ShapeDtypeStruct((1, n), jnp.float32),
    )(inter, biasg.reshape(1, n), global_bias.reshape(1, 1), W1,
      b1.reshape(1, 128), W2, b2.reshape(1, 64), W3, b3.reshape(1, 1))
    return out.reshape(n)


def kernel(user_ids, pos_item_ids, user_emb, item_emb, user_bias, item_bias,
           global_bias, W1, b1, W2, b2, W3, b3):
    uid = user_ids.astype(jnp.int32)
    iid = pos_item_ids.astype(jnp.int32)
    ub1 = user_bias.reshape(-1)
    ib1 = item_bias.reshape(-1)
    halves = []
    for h in range(2):
        s = slice(h * HB, (h + 1) * HB)
        inter, biasg = _sc_gather_half(uid[s], iid[s], user_emb, item_emb,
                                       ub1, ib1)
        halves.append(_tc_call(inter, biasg, global_bias, W1, b1, W2, b2,
                               W3, b3))
    return jnp.concatenate(halves)
